# Initial kernel scaffold; baseline (speedup 1.0000x reference)
import jax
import jax.numpy as jnp
from jax.experimental import pallas as pl


def kernel(hidden_states, gate_kernel, W1, b1, W2, b2, Ws1, bs1, Ws2, bs2):
    raise NotImplementedError("write your pallas kernel here")


# speedup vs baseline: 1.1766x; 1.1766x over previous
import functools

import jax
import jax.numpy as jnp
from jax.experimental import pallas as pl

B, S, H = 1, 2048, 1024
E, TOPK, DFF = 8, 2, 1024
T = B * S
NE = E + 1


def _gate_body(x_ref, gk_ref, wmask_ref):
    x = x_ref[...]
    gk = gk_ref[...]
    logits = jax.lax.dot_general(
        x, gk, (((1,), (0,)), ((), ())), preferred_element_type=jnp.float32
    )
    m = jnp.max(logits, axis=-1, keepdims=True)
    ex = jnp.exp(logits - m)
    scores = ex / jnp.sum(ex, axis=-1, keepdims=True)

    neg = jnp.float32(-1.0)
    best1 = jnp.full((T, 1), neg, jnp.float32)
    idx1 = jnp.zeros((T, 1), jnp.int32)
    for e in range(E):
        s = scores[:, e : e + 1]
        upd = s > best1
        idx1 = jnp.where(upd, e, idx1)
        best1 = jnp.where(upd, s, best1)
    best2 = jnp.full((T, 1), neg, jnp.float32)
    idx2 = jnp.zeros((T, 1), jnp.int32)
    for e in range(E):
        s = scores[:, e : e + 1]
        upd = jnp.logical_and(idx1 != e, s > best2)
        idx2 = jnp.where(upd, e, idx2)
        best2 = jnp.where(upd, s, best2)
    denom = best1 + best2 + jnp.float32(1e-20)
    w1 = best1 / denom
    w2 = best2 / denom

    cols = jax.lax.broadcasted_iota(jnp.int32, (T, NE), 1)
    wmask = jnp.where(cols == idx1, w1, 0.0) + jnp.where(cols == idx2, w2, 0.0)
    wmask = jnp.where(cols == E, 1.0, wmask)
    wmask_ref[...] = wmask


def _moe_body(x_ref, w1_ref, b1_ref, w2_ref, b2_ref, wm_ref, out_ref):
    e = pl.program_id(0)
    x = x_ref[...]
    h = jax.lax.dot_general(
        x, w1_ref[0], (((1,), (0,)), ((), ())), preferred_element_type=jnp.float32
    )
    h = jax.nn.gelu(h + b1_ref[0])
    o = jax.lax.dot_general(
        h, w2_ref[0], (((1,), (0,)), ((), ())), preferred_element_type=jnp.float32
    )
    o = o + b2_ref[0]
    cols = jax.lax.broadcasted_iota(jnp.int32, (T, NE), 1)
    wcol = jnp.sum(jnp.where(cols == e, wm_ref[...], 0.0), axis=1, keepdims=True)
    contrib = o * wcol

    @pl.when(e == 0)
    def _():
        out_ref[...] = contrib

    @pl.when(e != 0)
    def _():
        out_ref[...] = out_ref[...] + contrib


def kernel(hidden_states, gate_kernel, W1, b1, W2, b2, Ws1, bs1, Ws2, bs2):
    flat = hidden_states.reshape(T, H)
    gk_t = gate_kernel.T

    wmask = pl.pallas_call(
        _gate_body,
        out_shape=jax.ShapeDtypeStruct((T, NE), jnp.float32),
    )(flat, gk_t)

    W1a = jnp.concatenate([W1, Ws1[None]], axis=0)
    b1a = jnp.concatenate([b1, bs1[None]], axis=0).reshape(NE, 1, DFF)
    W2a = jnp.concatenate([W2, Ws2[None]], axis=0)
    b2a = jnp.concatenate([b2, bs2[None]], axis=0).reshape(NE, 1, H)

    y = pl.pallas_call(
        _moe_body,
        grid=(NE,),
        in_specs=[
            pl.BlockSpec((T, H), lambda e: (0, 0)),
            pl.BlockSpec((1, H, DFF), lambda e: (e, 0, 0)),
            pl.BlockSpec((1, 1, DFF), lambda e: (e, 0, 0)),
            pl.BlockSpec((1, DFF, H), lambda e: (e, 0, 0)),
            pl.BlockSpec((1, 1, H), lambda e: (e, 0, 0)),
            pl.BlockSpec((T, NE), lambda e: (0, 0)),
        ],
        out_specs=pl.BlockSpec((T, H), lambda e: (0, 0)),
        out_shape=jax.ShapeDtypeStruct((T, H), jnp.float32),
    )(flat, W1a, b1a, W2a, b2a, wmask)

    return y.reshape(B, S, H)
